# Initial kernel scaffold; baseline (speedup 1.0000x reference)
import jax
import jax.numpy as jnp
from jax.experimental import pallas as pl


def kernel(x, edge_index_connections, edge_index_destinations, W1l, W1r, W2l, W2r, W3l, W3r, W4l, W4r, g1, b1, g2, b2, g3, b3, g4, b4):
    raise NotImplementedError("write your pallas kernel here")


# speedup vs baseline: 12.3398x; 12.3398x over previous
import functools

import numpy as np
import jax
import jax.numpy as jnp
from jax import lax
from jax.experimental import pallas as pl
from jax.experimental.pallas import tpu as pltpu
from jax.experimental.pallas import tpu_sc as plsc

N_NODES = 100000
H = 8
PAD_ROWS = 96
NPAD = N_NODES + PAD_ROWS
ROWS_PER_TILE = NPAD // 16
R = N_NODES // 16
NC = 2
NS = 16
NW = NC * NS
CHUNK = 128

E_CONN = 6400000
E_DEST = 100000
E_CONN_PAD = ((E_CONN + NW * CHUNK - 1) // (NW * CHUNK)) * (NW * CHUNK)
E_DEST_PAD = ((E_DEST + NW * CHUNK - 1) // (NW * CHUNK)) * (NW * CHUNK)


def _make_scatter(e_pad):
    edges_per_tile = e_pad // NW
    n_chunks = edges_per_tile // CHUNK
    mesh = plsc.VectorSubcoreMesh(core_axis_name="c", subcore_axis_name="s")
    scratch = [
        pltpu.VMEM((CHUNK,), jnp.int32),
        pltpu.VMEM((CHUNK,), jnp.int32),
        pltpu.VMEM((CHUNK, H), jnp.float32),
        pltpu.VMEM_SHARED((NPAD, H), jnp.float32),
    ]

    def body(table_hbm, src_hbm, dst_hbm, zeros8_hbm, agg_out,
             idx_s, idx_d, rows, agg_sh):
        cid = lax.axis_index("c")
        sid = lax.axis_index("s")
        wid = cid * NS + sid
        r0 = sid * ROWS_PER_TILE
        pltpu.sync_copy(zeros8_hbm.at[pl.ds(r0, ROWS_PER_TILE)],
                        agg_sh.at[pl.ds(r0, ROWS_PER_TILE)])
        plsc.subcore_barrier()

        base = wid * edges_per_tile

        def step(i, carry):
            off = base + i * CHUNK
            pltpu.sync_copy(src_hbm.at[pl.ds(off, CHUNK)], idx_s)
            pltpu.sync_copy(dst_hbm.at[pl.ds(off, CHUNK)], idx_d)
            pltpu.sync_copy(table_hbm.at[idx_s], rows)
            pltpu.sync_copy(rows, agg_sh.at[idx_d], add=True)
            return carry

        lax.fori_loop(0, n_chunks, step, 0)
        plsc.subcore_barrier()
        pltpu.sync_copy(agg_sh.at[pl.ds(r0, ROWS_PER_TILE)],
                        agg_out.at[cid, pl.ds(r0, ROWS_PER_TILE)])

    return pl.kernel(body, mesh=mesh,
                     out_type=jax.ShapeDtypeStruct((NC, NPAD, H), jnp.float32),
                     compiler_params=pltpu.CompilerParams(
                         use_tc_tiling_on_sc=False),
                     scratch_types=scratch)


def _dense_body(a_ref, c_ref, h_ref, wl_ref, wr_ref, ok_ref, sp_ref,
                g_ref, b_ref, out_ref):
    cnt = c_ref[0] + c_ref[1]
    inv = 1.0 / jnp.maximum(cnt, 1.0)
    mean = (a_ref[0] + a_ref[1]) * inv
    o = (jnp.dot(mean, wl_ref[...], preferred_element_type=jnp.float32)
         + jnp.dot(h_ref[...], wr_ref[...], preferred_element_type=jnp.float32))
    gs = jnp.dot(o * o, ok_ref[...], preferred_element_type=jnp.float32)
    o = o / jnp.maximum(jnp.sqrt(gs), 1e-12)
    o = jnp.maximum(o, 0.0)
    s1 = jnp.dot(jnp.sum(o, axis=0, keepdims=True), sp_ref[...],
                 preferred_element_type=jnp.float32) * (1.0 / N_NODES)
    cent = o - s1
    var = jnp.dot(jnp.sum(cent * cent, axis=0, keepdims=True), sp_ref[...],
                  preferred_element_type=jnp.float32) * (1.0 / N_NODES)
    out_ref[...] = cent / jnp.sqrt(var + 1e-5) * g_ref[...] + b_ref[...]


_dense = pl.pallas_call(
    _dense_body,
    out_shape=jax.ShapeDtypeStruct((R, 128), jnp.float32),
)


def _kron_w(w):
    wt = jnp.zeros((H, H), jnp.float32).at[: w.shape[1], :].set(w.T)
    return jnp.kron(jnp.eye(16, dtype=jnp.float32), wt)


def _pad_edges(idx, e_pad):
    fill = jnp.full((e_pad - idx.shape[0],), N_NODES, jnp.int32)
    return jnp.concatenate([idx, fill])


def _pack_agg(agg):
    return agg[:, :N_NODES, :].reshape(NC, R, 128)


def _unpack_h(hpacked):
    h8 = hpacked.reshape(N_NODES, H)
    return jnp.concatenate([h8, jnp.zeros((PAD_ROWS, H), jnp.float32)])


def kernel(x, edge_index_connections, edge_index_destinations,
           W1l, W1r, W2l, W2r, W3l, W3r, W4l, W4r,
           g1, b1, g2, b2, g3, b3, g4, b4):
    f32 = jnp.float32
    src_c = _pad_edges(edge_index_connections[0], E_CONN_PAD)
    dst_c = _pad_edges(edge_index_connections[1], E_CONN_PAD)
    src_d = _pad_edges(edge_index_destinations[0], E_DEST_PAD)
    dst_d = _pad_edges(edge_index_destinations[1], E_DEST_PAD)
    zeros8 = jnp.zeros((NPAD, H), f32)

    xpad = (jnp.zeros((NPAD, H), f32)
            .at[:N_NODES, : x.shape[1]].set(x)
            .at[:, 7].set(1.0))
    ones_tab = jnp.ones((NPAD, H), f32)

    onesk = jnp.kron(jnp.eye(16, dtype=f32), jnp.ones((H, H), f32))
    sump = jnp.kron(jnp.ones((16, 16), f32), jnp.eye(H, dtype=f32))

    scat_c = _make_scatter(E_CONN_PAD)
    scat_d = _make_scatter(E_DEST_PAD)
    cnt_d_p = _pack_agg(scat_d(ones_tab, src_d, dst_d, zeros8))

    def dense(agg, cntp, htab, Wl, Wr, g, b):
        return _dense(_pack_agg(agg), cntp,
                      htab[:N_NODES].reshape(R, 128),
                      _kron_w(Wl), _kron_w(Wr), onesk, sump,
                      jnp.tile(g, 16)[None, :], jnp.tile(b, 16)[None, :])

    agg = scat_c(xpad, src_c, dst_c, zeros8)
    cnt_c_p = jnp.repeat(agg[:, :N_NODES, 7:8], H, axis=2).reshape(NC, R, 128)
    h = dense(agg, cnt_c_p, xpad, W1l, W1r, g1, b1)
    htab = _unpack_h(h)

    agg = scat_c(htab, src_c, dst_c, zeros8)
    h = dense(agg, cnt_c_p, htab, W4l, W4r, g2, b2)
    htab = _unpack_h(h)

    agg = scat_d(htab, src_d, dst_d, zeros8)
    h = dense(agg, cnt_d_p, htab, W2l, W2r, g3, b3)
    htab = _unpack_h(h)

    for _ in range(2):
        agg = scat_c(htab, src_c, dst_c, zeros8)
        h = dense(agg, cnt_c_p, htab, W3l, W3r, g4, b4)
        htab = _unpack_h(h)

    return h.reshape(N_NODES, H)


# speedup vs baseline: 36.7793x; 2.9805x over previous
import functools

import numpy as np
import jax
import jax.numpy as jnp
from jax import lax
from jax.experimental import pallas as pl
from jax.experimental.pallas import tpu as pltpu
from jax.experimental.pallas import tpu_sc as plsc

N_NODES = 100000
H = 8
PAD_ROWS = 96
NPAD = N_NODES + PAD_ROWS
ROWS_PER_TILE = NPAD // 16
R = N_NODES // 16
NC = 2
NS = 16
NW = NC * NS
CHUNK = 128

E_CONN = 6400000
E_DEST = 100000
E_ALIGN = NW * CHUNK * 8
E_CONN_PAD = ((E_CONN + E_ALIGN - 1) // E_ALIGN) * E_ALIGN
E_DEST_PAD = ((E_DEST + E_ALIGN - 1) // E_ALIGN) * E_ALIGN


K = 8
OUTER_E = K * CHUNK


def _make_scatter(e_pad):
    edges_per_tile = e_pad // NW
    n_outer = edges_per_tile // OUTER_E
    rows_per_iter = OUTER_E // CHUNK
    mesh = plsc.VectorSubcoreMesh(core_axis_name="c", subcore_axis_name="s")
    scratch = [
        pltpu.VMEM((3, K, CHUNK), jnp.int32),
        pltpu.VMEM((3, K, CHUNK), jnp.int32),
        pltpu.VMEM((2, K, CHUNK, H), jnp.float32),
        pltpu.VMEM_SHARED((NPAD, H), jnp.float32),
        pltpu.SemaphoreType.DMA,
        pltpu.SemaphoreType.DMA,
        pltpu.SemaphoreType.DMA,
        pltpu.SemaphoreType.DMA,
    ]

    def body(table_hbm, src_hbm, dst_hbm, zeros8_hbm, agg_out,
             idx_s, idx_d, rows, agg_sh, sem_is, sem_id, sem_g, sem_s):
        cid = lax.axis_index("c")
        sid = lax.axis_index("s")
        wid = cid * NS + sid
        r0 = sid * ROWS_PER_TILE
        pltpu.sync_copy(zeros8_hbm.at[pl.ds(r0, ROWS_PER_TILE)],
                        agg_sh.at[pl.ds(r0, ROWS_PER_TILE)])
        plsc.subcore_barrier()

        base_rows = wid * (edges_per_tile // CHUNK)

        def fire_idx(i, slot):
            ro = base_rows + i * rows_per_iter
            pltpu.async_copy(src_hbm.at[pl.ds(ro, K)], idx_s.at[slot], sem_is)
            pltpu.async_copy(dst_hbm.at[pl.ds(ro, K)], idx_d.at[slot], sem_id)

        def drain_scatters(b, n):
            for j in range(n):
                pltpu.make_async_copy(rows.at[b, j % K],
                                      agg_sh.at[pl.ds(0, CHUNK)],
                                      sem_s).wait()

        fire_idx(0, 0)

        def outer(i, carry):
            b3 = lax.rem(i, 3)
            b2 = lax.rem(i, 2)
            @pl.when(i >= 2)
            def _():
                drain_scatters(b2, K)
            pltpu.make_async_copy(src_hbm.at[pl.ds(0, K)], idx_s.at[b3],
                                  sem_is).wait()
            pltpu.make_async_copy(dst_hbm.at[pl.ds(0, K)], idx_d.at[b3],
                                  sem_id).wait()
            for j in range(K):
                pltpu.async_copy(table_hbm.at[idx_s.at[b3, j]],
                                 rows.at[b2, j], sem_g)
            @pl.when(i + 1 < n_outer)
            def _():
                fire_idx(i + 1, lax.rem(i + 1, 3))
            for j in range(K):
                pltpu.make_async_copy(table_hbm.at[pl.ds(0, CHUNK)],
                                      rows.at[b2, j], sem_g).wait()
                pltpu.async_copy(rows.at[b2, j], agg_sh.at[idx_d.at[b3, j]],
                                 sem_s, add=True)
            return carry

        lax.fori_loop(0, n_outer, outer, 0)
        drain_scatters(0, K)
        drain_scatters(1, K)
        plsc.subcore_barrier()
        pltpu.sync_copy(agg_sh.at[pl.ds(r0, ROWS_PER_TILE)],
                        agg_out.at[cid, pl.ds(r0, ROWS_PER_TILE)])

    return pl.kernel(body, mesh=mesh,
                     out_type=jax.ShapeDtypeStruct((NC, NPAD, H), jnp.float32),
                     compiler_params=pltpu.CompilerParams(
                         use_tc_tiling_on_sc=False),
                     scratch_types=scratch)


def _dense_body(a_ref, c_ref, h_ref, wl_ref, wr_ref, ok_ref, sp_ref,
                g_ref, b_ref, out_ref):
    cnt = c_ref[0] + c_ref[1]
    mean = (a_ref[0] + a_ref[1]) / jnp.maximum(cnt, 1.0)
    o = (jnp.dot(mean, wl_ref[...], preferred_element_type=jnp.float32)
         + jnp.dot(h_ref[...], wr_ref[...], preferred_element_type=jnp.float32))
    gs = jnp.dot(o * o, ok_ref[...], preferred_element_type=jnp.float32)
    o = o / jnp.maximum(jnp.sqrt(gs), 1e-12)
    o = jnp.maximum(o, 0.0)
    s1 = jnp.dot(jnp.sum(o, axis=0, keepdims=True), sp_ref[...],
                 preferred_element_type=jnp.float32) * (1.0 / N_NODES)
    cent = o - s1
    var = jnp.dot(jnp.sum(cent * cent, axis=0, keepdims=True), sp_ref[...],
                  preferred_element_type=jnp.float32) * (1.0 / N_NODES)
    out_ref[...] = cent / jnp.sqrt(var + 1e-5) * g_ref[...] + b_ref[...]


_dense = pl.pallas_call(
    _dense_body,
    out_shape=jax.ShapeDtypeStruct((R, 128), jnp.float32),
)


def _kron_w(w):
    wt = jnp.zeros((H, H), jnp.float32).at[: w.shape[1], :].set(w.T)
    return jnp.kron(jnp.eye(16, dtype=jnp.float32), wt)


def _pad_edges(idx, e_pad):
    fill = jnp.full((e_pad - idx.shape[0],), N_NODES, jnp.int32)
    return jnp.concatenate([idx, fill]).reshape(e_pad // CHUNK, CHUNK)


def _pack_agg(agg):
    return agg[:, :N_NODES, :].reshape(NC, R, 128)


def _unpack_h(hpacked):
    h8 = hpacked.reshape(N_NODES, H)
    return jnp.concatenate([h8, jnp.zeros((PAD_ROWS, H), jnp.float32)])


def kernel(x, edge_index_connections, edge_index_destinations,
           W1l, W1r, W2l, W2r, W3l, W3r, W4l, W4r,
           g1, b1, g2, b2, g3, b3, g4, b4):
    f32 = jnp.float32
    src_c = _pad_edges(edge_index_connections[0], E_CONN_PAD)
    dst_c = _pad_edges(edge_index_connections[1], E_CONN_PAD)
    src_d = _pad_edges(edge_index_destinations[0], E_DEST_PAD)
    dst_d = _pad_edges(edge_index_destinations[1], E_DEST_PAD)
    zeros8 = jnp.zeros((NPAD, H), f32)

    xpad = (jnp.zeros((NPAD, H), f32)
            .at[:N_NODES, : x.shape[1]].set(x)
            .at[:, 7].set(1.0))
    ones_tab = jnp.ones((NPAD, H), f32)

    onesk = jnp.kron(jnp.eye(16, dtype=f32), jnp.ones((H, H), f32))
    sump = jnp.kron(jnp.ones((16, 16), f32), jnp.eye(H, dtype=f32))

    scat_c = _make_scatter(E_CONN_PAD)
    scat_d = _make_scatter(E_DEST_PAD)
    cnt_d_p = _pack_agg(scat_d(ones_tab, src_d, dst_d, zeros8))

    def dense(agg, cntp, htab, Wl, Wr, g, b):
        return _dense(_pack_agg(agg), cntp,
                      htab[:N_NODES].reshape(R, 128),
                      _kron_w(Wl), _kron_w(Wr), onesk, sump,
                      jnp.tile(g, 16)[None, :], jnp.tile(b, 16)[None, :])

    agg = scat_c(xpad, src_c, dst_c, zeros8)
    cnt_c_p = jnp.repeat(agg[:, :N_NODES, 7:8], H, axis=2).reshape(NC, R, 128)
    h = dense(agg, cnt_c_p, xpad, W1l, W1r, g1, b1)
    htab = _unpack_h(h)

    agg = scat_c(htab, src_c, dst_c, zeros8)
    h = dense(agg, cnt_c_p, htab, W4l, W4r, g2, b2)
    htab = _unpack_h(h)

    agg = scat_d(htab, src_d, dst_d, zeros8)
    h = dense(agg, cnt_d_p, htab, W2l, W2r, g3, b3)
    htab = _unpack_h(h)

    for _ in range(2):
        agg = scat_c(htab, src_c, dst_c, zeros8)
        h = dense(agg, cnt_c_p, htab, W3l, W3r, g4, b4)
        htab = _unpack_h(h)

    return h.reshape(N_NODES, H)


# speedup vs baseline: 64.7851x; 1.7615x over previous
import functools

import numpy as np
import jax
import jax.numpy as jnp
from jax import lax
from jax.experimental import pallas as pl
from jax.experimental.pallas import tpu as pltpu
from jax.experimental.pallas import tpu_sc as plsc

N_NODES = 100000
H = 8
PAD_ROWS = 96
NPAD = N_NODES + PAD_ROWS
ROWS_PER_TILE = NPAD // 16
R = N_NODES // 16
RP = NPAD // 16
NC = 2
NS = 16
NW = NC * NS
CHUNK = 128

E_CONN = 6400000
E_DEST = 100000
E_ALIGN = NW * CHUNK * 8
E_CONN_PAD = ((E_CONN + E_ALIGN - 1) // E_ALIGN) * E_ALIGN
E_DEST_PAD = ((E_DEST + E_ALIGN - 1) // E_ALIGN) * E_ALIGN


K = 8
OUTER_E = K * CHUNK


def _make_scatter(e_pad):
    edges_per_tile = e_pad // NW
    n_outer = edges_per_tile // OUTER_E
    rows_per_iter = OUTER_E // CHUNK
    mesh = plsc.VectorSubcoreMesh(core_axis_name="c", subcore_axis_name="s")
    scratch = [
        pltpu.VMEM((3, K, CHUNK), jnp.int32),
        pltpu.VMEM((3, K, CHUNK), jnp.int32),
        pltpu.VMEM((2, K, CHUNK, H), jnp.float32),
        pltpu.VMEM_SHARED((NPAD, H), jnp.float32),
        pltpu.SemaphoreType.DMA,
        pltpu.SemaphoreType.DMA,
        pltpu.SemaphoreType.DMA,
        pltpu.SemaphoreType.DMA,
    ]

    def body(table_hbm, src_hbm, dst_hbm, zeros8_hbm, agg_out,
             idx_s, idx_d, rows, agg_sh, sem_is, sem_id, sem_g, sem_s):
        cid = lax.axis_index("c")
        sid = lax.axis_index("s")
        wid = cid * NS + sid
        r0 = sid * ROWS_PER_TILE
        pltpu.sync_copy(zeros8_hbm.at[pl.ds(r0, ROWS_PER_TILE)],
                        agg_sh.at[pl.ds(r0, ROWS_PER_TILE)])
        plsc.subcore_barrier()

        base_rows = wid * (edges_per_tile // CHUNK)

        def fire_idx(i, slot):
            ro = base_rows + i * rows_per_iter
            pltpu.async_copy(src_hbm.at[pl.ds(ro, K)], idx_s.at[slot], sem_is)
            pltpu.async_copy(dst_hbm.at[pl.ds(ro, K)], idx_d.at[slot], sem_id)

        def drain_scatters(b, n):
            for j in range(n):
                pltpu.make_async_copy(rows.at[b, j % K],
                                      agg_sh.at[pl.ds(0, CHUNK)],
                                      sem_s).wait()

        fire_idx(0, 0)

        def outer(i, carry):
            b3 = lax.rem(i, 3)
            b2 = lax.rem(i, 2)
            @pl.when(i >= 2)
            def _():
                drain_scatters(b2, K)
            pltpu.make_async_copy(src_hbm.at[pl.ds(0, K)], idx_s.at[b3],
                                  sem_is).wait()
            pltpu.make_async_copy(dst_hbm.at[pl.ds(0, K)], idx_d.at[b3],
                                  sem_id).wait()
            for j in range(K):
                pltpu.async_copy(table_hbm.at[idx_s.at[b3, j]],
                                 rows.at[b2, j], sem_g)
            @pl.when(i + 1 < n_outer)
            def _():
                fire_idx(i + 1, lax.rem(i + 1, 3))
            for j in range(K):
                pltpu.make_async_copy(table_hbm.at[pl.ds(0, CHUNK)],
                                      rows.at[b2, j], sem_g).wait()
                pltpu.async_copy(rows.at[b2, j], agg_sh.at[idx_d.at[b3, j]],
                                 sem_s, add=True)
            return carry

        lax.fori_loop(0, n_outer, outer, 0)
        drain_scatters(0, K)
        drain_scatters(1, K)
        plsc.subcore_barrier()
        pltpu.sync_copy(agg_sh.at[pl.ds(r0, ROWS_PER_TILE)],
                        agg_out.at[cid, pl.ds(r0, ROWS_PER_TILE)])

    return pl.kernel(body, mesh=mesh,
                     out_type=jax.ShapeDtypeStruct((NC, NPAD, H), jnp.float32),
                     compiler_params=pltpu.CompilerParams(
                         use_tc_tiling_on_sc=False),
                     scratch_types=scratch)


def _make_scatter_l1(e_pad_c, e_pad_d):
    edges_per_tile = e_pad_c // NW
    n_outer = edges_per_tile // OUTER_E
    rows_per_iter = OUTER_E // CHUNK
    drows = e_pad_d // CHUNK // NW
    mesh = plsc.VectorSubcoreMesh(core_axis_name="c", subcore_axis_name="s")
    scratch = [
        pltpu.VMEM((3, K, CHUNK), jnp.int32),
        pltpu.VMEM((3, K, CHUNK), jnp.int32),
        pltpu.VMEM((2, K, CHUNK, H), jnp.float32),
        pltpu.VMEM((drows, CHUNK), jnp.int32),
        pltpu.VMEM((CHUNK, H), jnp.float32),
        pltpu.VMEM_SHARED((NPAD, H), jnp.float32),
        pltpu.VMEM_SHARED((NPAD, H), jnp.float32),
        pltpu.SemaphoreType.DMA,
        pltpu.SemaphoreType.DMA,
        pltpu.SemaphoreType.DMA,
        pltpu.SemaphoreType.DMA,
        pltpu.SemaphoreType.DMA,
    ]

    def body(table_hbm, src_hbm, dst_hbm, dstd_hbm, zeros8_hbm, ones8_hbm,
             agg_out, cntd_out, idx_s, idx_d, rows, idx_dd, ones_v,
             agg_sh, cnt_sh, sem_is, sem_id, sem_g, sem_s, sem_cs):
        cid = lax.axis_index("c")
        sid = lax.axis_index("s")
        wid = cid * NS + sid
        r0 = sid * ROWS_PER_TILE
        pltpu.sync_copy(zeros8_hbm.at[pl.ds(r0, ROWS_PER_TILE)],
                        agg_sh.at[pl.ds(r0, ROWS_PER_TILE)])
        pltpu.sync_copy(zeros8_hbm.at[pl.ds(r0, ROWS_PER_TILE)],
                        cnt_sh.at[pl.ds(r0, ROWS_PER_TILE)])
        pltpu.sync_copy(ones8_hbm, ones_v)
        plsc.subcore_barrier()

        dbase = wid * drows
        pltpu.sync_copy(dstd_hbm.at[pl.ds(dbase, drows)], idx_dd)
        for grp in range(0, drows, 16):
            n = min(16, drows - grp)
            for j in range(n):
                pltpu.async_copy(ones_v, cnt_sh.at[idx_dd.at[grp + j]],
                                 sem_cs, add=True)
            for j in range(n):
                pltpu.make_async_copy(ones_v, cnt_sh.at[pl.ds(0, CHUNK)],
                                      sem_cs).wait()

        base_rows = wid * (edges_per_tile // CHUNK)

        def fire_idx(i, slot):
            ro = base_rows + i * rows_per_iter
            pltpu.async_copy(src_hbm.at[pl.ds(ro, K)], idx_s.at[slot], sem_is)
            pltpu.async_copy(dst_hbm.at[pl.ds(ro, K)], idx_d.at[slot], sem_id)

        def drain_scatters(b, n):
            for j in range(n):
                pltpu.make_async_copy(rows.at[b, j % K],
                                      agg_sh.at[pl.ds(0, CHUNK)],
                                      sem_s).wait()

        fire_idx(0, 0)

        def outer(i, carry):
            b3 = lax.rem(i, 3)
            b2 = lax.rem(i, 2)
            @pl.when(i >= 2)
            def _():
                drain_scatters(b2, K)
            pltpu.make_async_copy(src_hbm.at[pl.ds(0, K)], idx_s.at[b3],
                                  sem_is).wait()
            pltpu.make_async_copy(dst_hbm.at[pl.ds(0, K)], idx_d.at[b3],
                                  sem_id).wait()
            for j in range(K):
                pltpu.async_copy(table_hbm.at[idx_s.at[b3, j]],
                                 rows.at[b2, j], sem_g)
            @pl.when(i + 1 < n_outer)
            def _():
                fire_idx(i + 1, lax.rem(i + 1, 3))
            for j in range(K):
                pltpu.make_async_copy(table_hbm.at[pl.ds(0, CHUNK)],
                                      rows.at[b2, j], sem_g).wait()
                pltpu.async_copy(rows.at[b2, j], agg_sh.at[idx_d.at[b3, j]],
                                 sem_s, add=True)
            return carry

        lax.fori_loop(0, n_outer, outer, 0)
        drain_scatters(0, K)
        drain_scatters(1, K)
        plsc.subcore_barrier()
        pltpu.sync_copy(agg_sh.at[pl.ds(r0, ROWS_PER_TILE)],
                        agg_out.at[cid, pl.ds(r0, ROWS_PER_TILE)])
        pltpu.sync_copy(cnt_sh.at[pl.ds(r0, ROWS_PER_TILE)],
                        cntd_out.at[cid, pl.ds(r0, ROWS_PER_TILE)])

    return pl.kernel(body, mesh=mesh,
                     out_type=(jax.ShapeDtypeStruct((NC, NPAD, H), jnp.float32),
                               jax.ShapeDtypeStruct((NC, NPAD, H), jnp.float32)),
                     compiler_params=pltpu.CompilerParams(
                         use_tc_tiling_on_sc=False),
                     scratch_types=scratch)


def _dense_body(a_ref, c_ref, h_ref, wl_ref, wr_ref, ok_ref, sp_ref,
                cs_ref, g_ref, b_ref, out_ref):
    f32 = jnp.float32
    cnt = jnp.dot(c_ref[0] + c_ref[1], cs_ref[...], preferred_element_type=f32)
    mean = (a_ref[0] + a_ref[1]) / jnp.maximum(cnt, 1.0)
    o = (jnp.dot(mean, wl_ref[...], preferred_element_type=f32)
         + jnp.dot(h_ref[...], wr_ref[...], preferred_element_type=f32))
    gs = jnp.dot(o * o, ok_ref[...], preferred_element_type=f32)
    o = o / jnp.maximum(jnp.sqrt(gs), 1e-12)
    o = jnp.maximum(o, 0.0)
    mask = jax.lax.broadcasted_iota(jnp.int32, (RP, 128), 0) < R
    s1 = jnp.dot(jnp.sum(o, axis=0, keepdims=True), sp_ref[...],
                 preferred_element_type=f32) * (1.0 / N_NODES)
    cent = o - s1
    centm = jnp.where(mask, cent, 0.0)
    var = jnp.dot(jnp.sum(centm * centm, axis=0, keepdims=True), sp_ref[...],
                  preferred_element_type=f32) * (1.0 / N_NODES)
    out = cent / jnp.sqrt(var + 1e-5) * g_ref[...] + b_ref[...]
    out_ref[...] = jnp.where(mask, out, 0.0)


_dense = pl.pallas_call(
    _dense_body,
    out_shape=jax.ShapeDtypeStruct((RP, 128), jnp.float32),
)


def _kron_w(w):
    wt = jnp.zeros((H, H), jnp.float32).at[: w.shape[1], :].set(w.T)
    return jnp.kron(jnp.eye(16, dtype=jnp.float32), wt)


def _pad_edges(idx, e_pad):
    fill = jnp.full((e_pad - idx.shape[0],), N_NODES, jnp.int32)
    return jnp.concatenate([idx, fill]).reshape(e_pad // CHUNK, CHUNK)


def kernel(x, edge_index_connections, edge_index_destinations,
           W1l, W1r, W2l, W2r, W3l, W3r, W4l, W4r,
           g1, b1, g2, b2, g3, b3, g4, b4):
    f32 = jnp.float32
    src_c = _pad_edges(edge_index_connections[0], E_CONN_PAD)
    dst_c = _pad_edges(edge_index_connections[1], E_CONN_PAD)
    src_d = _pad_edges(edge_index_destinations[0], E_DEST_PAD)
    dst_d = _pad_edges(edge_index_destinations[1], E_DEST_PAD)
    zeros8 = jnp.zeros((NPAD, H), f32)
    ones8 = jnp.ones((CHUNK, H), f32)

    xpad = (jnp.zeros((NPAD, H), f32)
            .at[:N_NODES, : x.shape[1]].set(x)
            .at[:, 7].set(1.0))

    onesk = jnp.kron(jnp.eye(16, dtype=f32), jnp.ones((H, H), f32))
    sump = jnp.kron(jnp.ones((16, 16), f32), jnp.eye(H, dtype=f32))
    e7 = jnp.zeros((H, H), f32).at[7, :].set(1.0)
    e0 = jnp.zeros((H, H), f32).at[0, :].set(1.0)
    csel7 = jnp.kron(jnp.eye(16, dtype=f32), e7)
    csel0 = jnp.kron(jnp.eye(16, dtype=f32), e0)

    scat_l1 = _make_scatter_l1(E_CONN_PAD, E_DEST_PAD)
    scat_c = _make_scatter(E_CONN_PAD)
    scat_d = _make_scatter(E_DEST_PAD)

    def pk(a):
        return a.reshape(NC, RP, 128)

    def dense(aggp, cntp, htab, Wl, Wr, csel, g, b):
        return _dense(aggp, cntp, htab.reshape(RP, 128),
                      _kron_w(Wl), _kron_w(Wr), onesk, sump, csel,
                      jnp.tile(g, 16)[None, :], jnp.tile(b, 16)[None, :])

    agg, cntd = scat_l1(xpad, src_c, dst_c, dst_d, zeros8, ones8)
    cnt_c_p = pk(agg)
    cnt_d_p = pk(cntd)
    h = dense(cnt_c_p, cnt_c_p, xpad, W1l, W1r, csel7, g1, b1)
    htab = h.reshape(NPAD, H)

    agg = scat_c(htab, src_c, dst_c, zeros8)
    h = dense(pk(agg), cnt_c_p, htab, W4l, W4r, csel7, g2, b2)
    htab = h.reshape(NPAD, H)

    agg = scat_d(htab, src_d, dst_d, zeros8)
    h = dense(pk(agg), cnt_d_p, htab, W2l, W2r, csel0, g3, b3)
    htab = h.reshape(NPAD, H)

    for _ in range(2):
        agg = scat_c(htab, src_c, dst_c, zeros8)
        h = dense(pk(agg), cnt_c_p, htab, W3l, W3r, csel7, g4, b4)
        htab = h.reshape(NPAD, H)

    return htab[:N_NODES]
